# baseline (speedup 1.0000x reference)
import functools

import jax
import jax.numpy as jnp
from jax import lax
from jax.experimental import pallas as pl
from jax.experimental.pallas import tpu as pltpu
from jax.experimental.pallas import tpu_sc as plsc

N = 10000
NP = 10240
E = 320000
D = 128
DC = 16
G = 64
NC = 2
NS = 16
NW = NC * NS
BPW = 80
NQ = 4
CK = BPW // NQ
EPAD = NW * BPW * 128
DUMMY = N
BR = 2048
GRID = NP // BR
RPS = NP // NS

_mesh = plsc.VectorSubcoreMesh(core_axis_name="c", subcore_axis_name="s",
                               num_cores=NC, num_subcores=NS)


@functools.partial(
    pl.kernel,
    out_type=jax.ShapeDtypeStruct((NC, NP, D), jnp.float32),
    mesh=_mesh,
    compiler_params=pltpu.CompilerParams(use_tc_tiling_on_sc=False),
    scratch_types=[
        pltpu.VMEM((CK, 128), jnp.int32),
        pltpu.VMEM((BPW, 128), jnp.int32),
        pltpu.VMEM((2, 128, D), jnp.float32),
        pltpu.VMEM_SHARED((NP, D), jnp.float32),
        pltpu.SemaphoreType.DMA,
        pltpu.SemaphoreType.DMA,
    ],
)
def _sc_agg128(y_hbm, src_hbm, dst_hbm, z_hbm, out_hbm,
               src_v, dst_v, rows_v, acc_sh, gsem, ssem):
    c = lax.axis_index("c")
    s = lax.axis_index("s")
    w = c * NS + s
    pltpu.sync_copy(z_hbm, acc_sh.at[pl.ds(s * RPS, RPS)])
    pltpu.sync_copy(dst_hbm.at[w], dst_v)
    plsc.subcore_barrier()

    @pl.loop(0, NQ)
    def _(q):
        pltpu.sync_copy(src_hbm.at[w, q], src_v)
        qb = q * CK
        pltpu.async_copy(y_hbm.at[src_v.at[0]], rows_v.at[0], gsem)

        @pl.loop(0, CK, step=2)
        def _(r):
            for b in range(2):
                jl = r + b
                pltpu.make_async_copy(
                    y_hbm.at[src_v.at[jl]], rows_v.at[b], gsem).wait()
                pltpu.async_copy(
                    rows_v.at[b], acc_sh.at[dst_v.at[qb + jl]], ssem,
                    add=True)

                @pl.when(jl >= 1)
                def _():
                    pltpu.make_async_copy(
                        rows_v.at[1 - b],
                        acc_sh.at[dst_v.at[qb + jl - 1]], ssem).wait()

                @pl.when(jl + 1 < CK)
                def _():
                    pltpu.async_copy(
                        y_hbm.at[src_v.at[jl + 1]], rows_v.at[1 - b], gsem)

        pltpu.make_async_copy(
            rows_v.at[(CK - 1) % 2],
            acc_sh.at[dst_v.at[qb + CK - 1]], ssem).wait()

    plsc.subcore_barrier()
    pltpu.sync_copy(acc_sh.at[pl.ds(s * RPS, RPS)],
                    out_hbm.at[c, pl.ds(s * RPS, RPS)])


@functools.partial(
    pl.kernel,
    out_type=jax.ShapeDtypeStruct((NC, NP, DC), jnp.float32),
    mesh=_mesh,
    compiler_params=pltpu.CompilerParams(use_tc_tiling_on_sc=False),
    scratch_types=[
        pltpu.VMEM((BPW, 128), jnp.int32),
        pltpu.VMEM((BPW, 128), jnp.int32),
        pltpu.VMEM((2, 128, DC), jnp.float32),
        pltpu.VMEM_SHARED((NP, DC), jnp.float32),
        pltpu.SemaphoreType.DMA,
    ],
)
def _sc_agg16(y_hbm, src_hbm, dst_hbm, z_hbm, out_hbm,
              src_v, dst_v, rows_v, acc_sh, gsem):
    c = lax.axis_index("c")
    s = lax.axis_index("s")
    w = c * NS + s
    pltpu.sync_copy(z_hbm, acc_sh.at[pl.ds(s * RPS, RPS)])
    pltpu.sync_copy(src_hbm.at[w], src_v)
    pltpu.sync_copy(dst_hbm.at[w], dst_v)
    plsc.subcore_barrier()

    pltpu.async_copy(y_hbm.at[src_v.at[0]], rows_v.at[0], gsem)
    pltpu.async_copy(y_hbm.at[src_v.at[1]], rows_v.at[1], gsem)

    @pl.loop(0, BPW, step=2)
    def _(j):
        for b in range(2):
            jj = j + b
            pltpu.make_async_copy(
                y_hbm.at[src_v.at[jj]], rows_v.at[b], gsem).wait()
            pltpu.sync_copy(rows_v.at[b], acc_sh.at[dst_v.at[jj]], add=True)

            @pl.when(jj + 2 < BPW)
            def _():
                pltpu.async_copy(
                    y_hbm.at[src_v.at[jj + 2]], rows_v.at[b], gsem)

    plsc.subcore_barrier()
    pltpu.sync_copy(acc_sh.at[pl.ds(s * RPS, RPS)],
                    out_hbm.at[c, pl.ds(s * RPS, RPS)])


@functools.partial(
    pl.kernel,
    out_type=jax.ShapeDtypeStruct((NC, NP, DC), jnp.float32),
    mesh=_mesh,
    compiler_params=pltpu.CompilerParams(use_tc_tiling_on_sc=False),
    scratch_types=[
        pltpu.VMEM((BPW, 128), jnp.int32),
        pltpu.VMEM((128, DC), jnp.float32),
        pltpu.VMEM_SHARED((NP, DC), jnp.float32),
    ],
)
def _sc_deg(dst_hbm, ones_hbm, z_hbm, out_hbm, dst_v, ones_v, deg_sh):
    c = lax.axis_index("c")
    s = lax.axis_index("s")
    w = c * NS + s
    pltpu.sync_copy(z_hbm, deg_sh.at[pl.ds(s * RPS, RPS)])
    pltpu.sync_copy(ones_hbm, ones_v)
    pltpu.sync_copy(dst_hbm.at[w], dst_v)
    plsc.subcore_barrier()

    @pl.loop(0, BPW)
    def _(j):
        pltpu.sync_copy(ones_v, deg_sh.at[dst_v.at[j]], add=True)

    plsc.subcore_barrier()
    pltpu.sync_copy(deg_sh.at[pl.ds(s * RPS, RPS)],
                    out_hbm.at[c, pl.ds(s * RPS, RPS)])


def _tc_first_body(x_ref, degp_ref, w_ref, y_ref, dinv_ref):
    dp = degp_ref[...]
    deg = dp[0, :, 0] + dp[1, :, 0] + 1.0
    dinv = lax.rsqrt(deg)
    xw = jnp.dot(x_ref[...], w_ref[...], preferred_element_type=jnp.float32)
    y_ref[...] = dinv[:, None] * xw
    dinv_ref[...] = jnp.broadcast_to(dinv[:, None], (BR, D))


def _tc_first(x, degp, w1):
    return pl.pallas_call(
        _tc_first_body,
        grid=(GRID,),
        in_specs=[
            pl.BlockSpec((BR, D), lambda i: (i, 0)),
            pl.BlockSpec((NC, BR, DC), lambda i: (0, i, 0)),
            pl.BlockSpec((D, D), lambda i: (0, 0)),
        ],
        out_specs=[
            pl.BlockSpec((BR, D), lambda i: (i, 0)),
            pl.BlockSpec((BR, D), lambda i: (i, 0)),
        ],
        out_shape=[
            jax.ShapeDtypeStruct((NP, D), jnp.float32),
            jax.ShapeDtypeStruct((NP, D), jnp.float32),
        ],
    )(x, degp, w1)


def _tc_mid_body(aggp_ref, y_ref, dinv_ref, b_ref, w_ref, o_ref, nout):
    ag = aggp_ref[...]
    dinv = dinv_ref[...]
    h = dinv * (ag[0] + ag[1] + y_ref[...]) + b_ref[...]
    h = jnp.maximum(h, 0.0)
    hw = jnp.dot(h, w_ref[...], preferred_element_type=jnp.float32)
    o_ref[...] = dinv[:, :nout] * hw


def _tc_mid(aggp, y, dinv, b, w):
    nout = w.shape[1]
    return pl.pallas_call(
        functools.partial(_tc_mid_body, nout=nout),
        grid=(GRID,),
        in_specs=[
            pl.BlockSpec((NC, BR, D), lambda i: (0, i, 0)),
            pl.BlockSpec((BR, D), lambda i: (i, 0)),
            pl.BlockSpec((BR, D), lambda i: (i, 0)),
            pl.BlockSpec((1, D), lambda i: (0, 0)),
            pl.BlockSpec((D, nout), lambda i: (0, 0)),
        ],
        out_specs=pl.BlockSpec((BR, nout), lambda i: (i, 0)),
        out_shape=jax.ShapeDtypeStruct((NP, nout), jnp.float32),
    )(aggp, y, dinv, b, w)


def _tc_pool_body(aggp_ref, y_ref, dinv_ref, b_ref, batch_ref, o_ref,
                  sums_ref, cnts_ref):
    i = pl.program_id(0)
    ag = aggp_ref[...]
    dinv = dinv_ref[...][:, :DC]
    h4 = dinv * (ag[0] + ag[1] + y_ref[...]) + b_ref[...]
    gid = jax.lax.broadcasted_iota(jnp.int32, (BR, G), 1)
    onehot = (batch_ref[...] == gid).astype(jnp.float32)
    part_s = lax.dot_general(onehot, h4, (((0,), (0,)), ((), ())),
                             preferred_element_type=jnp.float32)
    part_c = jnp.sum(onehot, axis=0)

    @pl.when(i == 0)
    def _():
        sums_ref[...] = part_s
        cnts_ref[...] = jnp.broadcast_to(part_c[:, None], (G, DC))

    @pl.when(i > 0)
    def _():
        sums_ref[...] += part_s
        cnts_ref[...] += jnp.broadcast_to(part_c[:, None], (G, DC))

    @pl.when(i == GRID - 1)
    def _():
        o_ref[...] = sums_ref[...] / jnp.maximum(cnts_ref[...], 1.0)


def _tc_pool(aggp, y, dinv, b, batch2d):
    return pl.pallas_call(
        _tc_pool_body,
        grid=(GRID,),
        in_specs=[
            pl.BlockSpec((NC, BR, DC), lambda i: (0, i, 0)),
            pl.BlockSpec((BR, DC), lambda i: (i, 0)),
            pl.BlockSpec((BR, D), lambda i: (i, 0)),
            pl.BlockSpec((1, DC), lambda i: (0, 0)),
            pl.BlockSpec((BR, 1), lambda i: (i, 0)),
        ],
        out_specs=pl.BlockSpec((G, DC), lambda i: (0, 0)),
        out_shape=jax.ShapeDtypeStruct((G, DC), jnp.float32),
        scratch_shapes=[
            pltpu.VMEM((G, DC), jnp.float32),
            pltpu.VMEM((G, DC), jnp.float32),
        ],
    )(aggp, y, dinv, b, batch2d)


def kernel(x, edge_index, batch, W1, b1, W2, b2, W3, b3, W4, b4):
    f32 = jnp.float32
    src = edge_index[0]
    dst = edge_index[1]
    npad = EPAD - E
    srcf = jnp.concatenate([src, jnp.zeros((npad,), jnp.int32)])
    dstf = jnp.concatenate([dst, jnp.full((npad,), DUMMY, jnp.int32)])
    srcB = srcf.reshape(NW, BPW, 128)
    srcQ = srcB.reshape(NW, NQ, CK, 128)
    dstB = dstf.reshape(NW, BPW, 128)

    xp = jnp.zeros((NP, D), f32).at[:N].set(x)
    batch2d = jnp.full((NP, 1), -1, jnp.int32).at[:N, 0].set(batch)

    w4p = jnp.zeros((D, DC), f32).at[:, :10].set(W4)
    b4p = jnp.zeros((1, DC), f32).at[0, :10].set(b4)

    z16 = jnp.zeros((RPS, DC), f32)
    z128 = jnp.zeros((RPS, D), f32)
    ones16 = jnp.ones((128, DC), f32)

    degp = _sc_deg(dstB, ones16, z16)
    y1, dinv = _tc_first(xp, degp, W1)

    agg1 = _sc_agg128(y1, srcQ, dstB, z128)
    y2 = _tc_mid(agg1, y1, dinv, b1.reshape(1, D), W2)

    agg2 = _sc_agg128(y2, srcQ, dstB, z128)
    y3 = _tc_mid(agg2, y2, dinv, b2.reshape(1, D), W3)

    agg3 = _sc_agg128(y3, srcQ, dstB, z128)
    y4 = _tc_mid(agg3, y3, dinv, b3.reshape(1, D), w4p)

    agg4 = _sc_agg16(y4, srcB, dstB, z16)
    out = _tc_pool(agg4, y4, dinv, b4p, batch2d)
    return out[:, :10]


# speedup vs baseline: 2.9606x; 2.9606x over previous
import functools

import jax
import jax.numpy as jnp
from jax import lax
from jax.experimental import pallas as pl
from jax.experimental.pallas import tpu as pltpu
from jax.experimental.pallas import tpu_sc as plsc

N = 10000
NP = 10240
E = 320000
D = 128
DH = 64
DC = 16
G = 64
NC = 2
NS = 16
NW = NC * NS
BPT = 160
BPW = 80
NQ = 4
CK = BPT // NQ
EPAD = NS * BPT * 128
DUMMY = N
BR = 2048
GRID = NP // BR
RPS = NP // NS

_mesh = plsc.VectorSubcoreMesh(core_axis_name="c", subcore_axis_name="s",
                               num_cores=NC, num_subcores=NS)


@functools.partial(
    pl.kernel,
    out_type=jax.ShapeDtypeStruct((NC, NP, DH), jnp.float32),
    mesh=_mesh,
    compiler_params=pltpu.CompilerParams(use_tc_tiling_on_sc=False),
    scratch_types=[
        pltpu.VMEM((CK, 128), jnp.int32),
        pltpu.VMEM((CK, 128), jnp.int32),
        pltpu.VMEM((4, 128, DH), jnp.float32),
        pltpu.VMEM_SHARED((NP, DH), jnp.float32),
        pltpu.VMEM_SHARED((NP, DH), jnp.float32),
        pltpu.SemaphoreType.DMA,
        pltpu.SemaphoreType.DMA,
    ],
)
def _sc_agg128(y_hbm, src_hbm, dst_hbm, z_hbm, out_hbm,
               src_v, dst_v, rows_v, y_sh, acc_sh, gsem, ssem):
    c = lax.axis_index("c")
    s = lax.axis_index("s")
    row0 = s * RPS
    pltpu.sync_copy(y_hbm.at[c, pl.ds(row0, RPS)], y_sh.at[pl.ds(row0, RPS)])
    pltpu.sync_copy(z_hbm, acc_sh.at[pl.ds(row0, RPS)])
    plsc.subcore_barrier()

    @pl.loop(0, NQ)
    def _(q):
        pltpu.sync_copy(src_hbm.at[s, q], src_v)
        pltpu.sync_copy(dst_hbm.at[s, q], dst_v)
        pltpu.async_copy(y_sh.at[src_v.at[0]], rows_v.at[0], gsem)
        pltpu.async_copy(y_sh.at[src_v.at[1]], rows_v.at[1], gsem)

        @pl.loop(0, CK, step=4)
        def _(r):
            for b in range(4):
                jl = r + b
                nx = (b + 2) % 4
                pltpu.make_async_copy(
                    y_sh.at[src_v.at[jl]], rows_v.at[b], gsem).wait()
                pltpu.async_copy(
                    rows_v.at[b], acc_sh.at[dst_v.at[jl]], ssem, add=True)

                @pl.when(jl >= 2)
                def _():
                    pltpu.make_async_copy(
                        rows_v.at[nx],
                        acc_sh.at[dst_v.at[jl - 2]], ssem).wait()

                @pl.when(jl + 2 < CK)
                def _():
                    pltpu.async_copy(
                        y_sh.at[src_v.at[jl + 2]], rows_v.at[nx], gsem)

        pltpu.make_async_copy(
            rows_v.at[2], acc_sh.at[dst_v.at[CK - 2]], ssem).wait()
        pltpu.make_async_copy(
            rows_v.at[3], acc_sh.at[dst_v.at[CK - 1]], ssem).wait()

    plsc.subcore_barrier()
    pltpu.sync_copy(acc_sh.at[pl.ds(row0, RPS)],
                    out_hbm.at[c, pl.ds(row0, RPS)])


@functools.partial(
    pl.kernel,
    out_type=jax.ShapeDtypeStruct((NC, NP, DC), jnp.float32),
    mesh=_mesh,
    compiler_params=pltpu.CompilerParams(use_tc_tiling_on_sc=False),
    scratch_types=[
        pltpu.VMEM((BPW, 128), jnp.int32),
        pltpu.VMEM((BPW, 128), jnp.int32),
        pltpu.VMEM((2, 128, DC), jnp.float32),
        pltpu.VMEM_SHARED((NP, DC), jnp.float32),
        pltpu.VMEM_SHARED((NP, DC), jnp.float32),
        pltpu.SemaphoreType.DMA,
    ],
)
def _sc_agg16(y_hbm, src_hbm, dst_hbm, z_hbm, out_hbm,
              src_v, dst_v, rows_v, y_sh, acc_sh, gsem):
    c = lax.axis_index("c")
    s = lax.axis_index("s")
    w = c * NS + s
    row0 = s * RPS
    pltpu.sync_copy(y_hbm.at[pl.ds(row0, RPS)], y_sh.at[pl.ds(row0, RPS)])
    pltpu.sync_copy(z_hbm, acc_sh.at[pl.ds(row0, RPS)])
    pltpu.sync_copy(src_hbm.at[w], src_v)
    pltpu.sync_copy(dst_hbm.at[w], dst_v)
    plsc.subcore_barrier()

    pltpu.async_copy(y_sh.at[src_v.at[0]], rows_v.at[0], gsem)
    pltpu.async_copy(y_sh.at[src_v.at[1]], rows_v.at[1], gsem)

    @pl.loop(0, BPW, step=2)
    def _(j):
        for b in range(2):
            jj = j + b
            pltpu.make_async_copy(
                y_sh.at[src_v.at[jj]], rows_v.at[b], gsem).wait()
            pltpu.sync_copy(rows_v.at[b], acc_sh.at[dst_v.at[jj]], add=True)

            @pl.when(jj + 2 < BPW)
            def _():
                pltpu.async_copy(
                    y_sh.at[src_v.at[jj + 2]], rows_v.at[b], gsem)

    plsc.subcore_barrier()
    pltpu.sync_copy(acc_sh.at[pl.ds(row0, RPS)],
                    out_hbm.at[c, pl.ds(row0, RPS)])


@functools.partial(
    pl.kernel,
    out_type=jax.ShapeDtypeStruct((NC, NP, DC), jnp.float32),
    mesh=_mesh,
    compiler_params=pltpu.CompilerParams(use_tc_tiling_on_sc=False),
    scratch_types=[
        pltpu.VMEM((BPW, 128), jnp.int32),
        pltpu.VMEM((128, DC), jnp.float32),
        pltpu.VMEM_SHARED((NP, DC), jnp.float32),
    ],
)
def _sc_deg(dst_hbm, ones_hbm, z_hbm, out_hbm, dst_v, ones_v, deg_sh):
    c = lax.axis_index("c")
    s = lax.axis_index("s")
    w = c * NS + s
    pltpu.sync_copy(z_hbm, deg_sh.at[pl.ds(s * RPS, RPS)])
    pltpu.sync_copy(ones_hbm, ones_v)
    pltpu.sync_copy(dst_hbm.at[w], dst_v)
    plsc.subcore_barrier()

    @pl.loop(0, BPW)
    def _(j):
        pltpu.sync_copy(ones_v, deg_sh.at[dst_v.at[j]], add=True)

    plsc.subcore_barrier()
    pltpu.sync_copy(deg_sh.at[pl.ds(s * RPS, RPS)],
                    out_hbm.at[c, pl.ds(s * RPS, RPS)])


def _tc_first_body(x_ref, degp_ref, w_ref, y_ref, dinv_ref):
    dp = degp_ref[...]
    deg = dp[0, :, 0] + dp[1, :, 0] + 1.0
    dinv = lax.rsqrt(deg)
    xw = jnp.dot(x_ref[...], w_ref[...], preferred_element_type=jnp.float32)
    y = dinv[:, None] * xw
    y_ref[0] = y[:, :DH]
    y_ref[1] = y[:, DH:]
    dinv_ref[...] = jnp.broadcast_to(dinv[:, None], (BR, D))


def _tc_first(x, degp, w1):
    return pl.pallas_call(
        _tc_first_body,
        grid=(GRID,),
        in_specs=[
            pl.BlockSpec((BR, D), lambda i: (i, 0)),
            pl.BlockSpec((NC, BR, DC), lambda i: (0, i, 0)),
            pl.BlockSpec((D, D), lambda i: (0, 0)),
        ],
        out_specs=[
            pl.BlockSpec((2, BR, DH), lambda i: (0, i, 0)),
            pl.BlockSpec((BR, D), lambda i: (i, 0)),
        ],
        out_shape=[
            jax.ShapeDtypeStruct((2, NP, DH), jnp.float32),
            jax.ShapeDtypeStruct((NP, D), jnp.float32),
        ],
    )(x, degp, w1)


def _tc_mid_body(agg_ref, y_ref, dinv_ref, b_ref, w_ref, o_ref, nout):
    ag = agg_ref[...]
    yy = y_ref[...]
    dinv = dinv_ref[...]
    pre = jnp.concatenate([ag[0] + yy[0], ag[1] + yy[1]], axis=1)
    h = jnp.maximum(dinv * pre + b_ref[...], 0.0)
    hw = jnp.dot(h, w_ref[...], preferred_element_type=jnp.float32)
    out = dinv[:, :nout] * hw
    if nout == D:
        o_ref[0] = out[:, :DH]
        o_ref[1] = out[:, DH:]
    else:
        o_ref[...] = out


def _tc_mid(agg, y, dinv, b, w):
    nout = w.shape[1]
    if nout == D:
        out_spec = pl.BlockSpec((2, BR, DH), lambda i: (0, i, 0))
        out_shape = jax.ShapeDtypeStruct((2, NP, DH), jnp.float32)
    else:
        out_spec = pl.BlockSpec((BR, nout), lambda i: (i, 0))
        out_shape = jax.ShapeDtypeStruct((NP, nout), jnp.float32)
    return pl.pallas_call(
        functools.partial(_tc_mid_body, nout=nout),
        grid=(GRID,),
        in_specs=[
            pl.BlockSpec((NC, BR, DH), lambda i: (0, i, 0)),
            pl.BlockSpec((2, BR, DH), lambda i: (0, i, 0)),
            pl.BlockSpec((BR, D), lambda i: (i, 0)),
            pl.BlockSpec((1, D), lambda i: (0, 0)),
            pl.BlockSpec((D, nout), lambda i: (0, 0)),
        ],
        out_specs=out_spec,
        out_shape=out_shape,
    )(agg, y, dinv, b, w)


def _tc_pool_body(aggp_ref, y_ref, dinv_ref, b_ref, batch_ref, o_ref,
                  sums_ref, cnts_ref):
    i = pl.program_id(0)
    ag = aggp_ref[...]
    dinv = dinv_ref[...][:, :DC]
    h4 = dinv * (ag[0] + ag[1] + y_ref[...]) + b_ref[...]
    gid = jax.lax.broadcasted_iota(jnp.int32, (BR, G), 1)
    onehot = (batch_ref[...] == gid).astype(jnp.float32)
    part_s = lax.dot_general(onehot, h4, (((0,), (0,)), ((), ())),
                             preferred_element_type=jnp.float32)
    part_c = jnp.sum(onehot, axis=0)

    @pl.when(i == 0)
    def _():
        sums_ref[...] = part_s
        cnts_ref[...] = jnp.broadcast_to(part_c[:, None], (G, DC))

    @pl.when(i > 0)
    def _():
        sums_ref[...] += part_s
        cnts_ref[...] += jnp.broadcast_to(part_c[:, None], (G, DC))

    @pl.when(i == GRID - 1)
    def _():
        o_ref[...] = sums_ref[...] / jnp.maximum(cnts_ref[...], 1.0)


def _tc_pool(aggp, y, dinv, b, batch2d):
    return pl.pallas_call(
        _tc_pool_body,
        grid=(GRID,),
        in_specs=[
            pl.BlockSpec((NC, BR, DC), lambda i: (0, i, 0)),
            pl.BlockSpec((BR, DC), lambda i: (i, 0)),
            pl.BlockSpec((BR, D), lambda i: (i, 0)),
            pl.BlockSpec((1, DC), lambda i: (0, 0)),
            pl.BlockSpec((BR, 1), lambda i: (i, 0)),
        ],
        out_specs=pl.BlockSpec((G, DC), lambda i: (0, 0)),
        out_shape=jax.ShapeDtypeStruct((G, DC), jnp.float32),
        scratch_shapes=[
            pltpu.VMEM((G, DC), jnp.float32),
            pltpu.VMEM((G, DC), jnp.float32),
        ],
    )(aggp, y, dinv, b, batch2d)


def kernel(x, edge_index, batch, W1, b1, W2, b2, W3, b3, W4, b4):
    f32 = jnp.float32
    src = edge_index[0]
    dst = edge_index[1]
    npad = EPAD - E
    srcf = jnp.concatenate([src, jnp.zeros((npad,), jnp.int32)])
    dstf = jnp.concatenate([dst, jnp.full((npad,), DUMMY, jnp.int32)])
    srcA = srcf.reshape(NS, NQ, CK, 128)
    dstA = dstf.reshape(NS, NQ, CK, 128)
    srcB = srcf.reshape(NW, BPW, 128)
    dstB = dstf.reshape(NW, BPW, 128)

    xp = jnp.zeros((NP, D), f32).at[:N].set(x)
    batch2d = jnp.full((NP, 1), -1, jnp.int32).at[:N, 0].set(batch)

    w4p = jnp.zeros((D, DC), f32).at[:, :10].set(W4)
    b4p = jnp.zeros((1, DC), f32).at[0, :10].set(b4)

    z16 = jnp.zeros((RPS, DC), f32)
    z64 = jnp.zeros((RPS, DH), f32)
    ones16 = jnp.ones((128, DC), f32)

    degp = _sc_deg(dstB, ones16, z16)
    y1, dinv = _tc_first(xp, degp, W1)

    agg1 = _sc_agg128(y1, srcA, dstA, z64)
    y2 = _tc_mid(agg1, y1, dinv, b1.reshape(1, D), W2)

    agg2 = _sc_agg128(y2, srcA, dstA, z64)
    y3 = _tc_mid(agg2, y2, dinv, b2.reshape(1, D), W3)

    agg3 = _sc_agg128(y3, srcA, dstA, z64)
    y4 = _tc_mid(agg3, y3, dinv, b3.reshape(1, D), w4p)

    agg4 = _sc_agg16(y4, srcB, dstB, z16)
    out = _tc_pool(agg4, y4, dinv, b4p, batch2d)
    return out[:, :10]


# speedup vs baseline: 3.3129x; 1.1190x over previous
import functools

import jax
import jax.numpy as jnp
from jax import lax
from jax.experimental import pallas as pl
from jax.experimental.pallas import tpu as pltpu
from jax.experimental.pallas import tpu_sc as plsc

N = 10000
NP = 10240
E = 320000
D = 128
DH = 64
DC = 16
G = 64
NC = 2
NS = 16
NW = NC * NS
BPT = 160
BPW = 80
NQ = 4
CK = BPT // NQ
EPAD = NS * BPT * 128
DUMMY = N
BR = 2048
GRID = NP // BR
RPS = NP // NS

_mesh = plsc.VectorSubcoreMesh(core_axis_name="c", subcore_axis_name="s",
                               num_cores=NC, num_subcores=NS)


@functools.partial(
    pl.kernel,
    out_type=jax.ShapeDtypeStruct((NP, D), jnp.float32),
    mesh=_mesh,
    compiler_params=pltpu.CompilerParams(use_tc_tiling_on_sc=False),
    scratch_types=[
        pltpu.VMEM((CK, 128), jnp.int32),
        pltpu.VMEM((CK, 128), jnp.int32),
        pltpu.VMEM((4, 128, DH), jnp.float32),
        pltpu.VMEM_SHARED((NP, DH), jnp.float32),
        pltpu.VMEM_SHARED((NP, DH), jnp.float32),
        pltpu.SemaphoreType.DMA,
        pltpu.SemaphoreType.DMA,
    ],
)
def _sc_agg128(y_hbm, src_hbm, dst_hbm, z_hbm, out_hbm,
               src_v, dst_v, rows_v, y_sh, acc_sh, gsem, ssem):
    c = lax.axis_index("c")
    s = lax.axis_index("s")
    row0 = s * RPS
    col0 = c * DH
    pltpu.sync_copy(y_hbm.at[pl.ds(row0, RPS), pl.ds(col0, DH)],
                    y_sh.at[pl.ds(row0, RPS)])
    pltpu.sync_copy(z_hbm, acc_sh.at[pl.ds(row0, RPS)])
    plsc.subcore_barrier()

    @pl.loop(0, NQ)
    def _(q):
        pltpu.sync_copy(src_hbm.at[s, q], src_v)
        pltpu.sync_copy(dst_hbm.at[s, q], dst_v)
        pltpu.async_copy(y_sh.at[src_v.at[0]], rows_v.at[0], gsem)
        pltpu.async_copy(y_sh.at[src_v.at[1]], rows_v.at[1], gsem)

        @pl.loop(0, CK, step=4)
        def _(r):
            for b in range(4):
                jl = r + b
                nx = (b + 2) % 4
                pltpu.make_async_copy(
                    y_sh.at[src_v.at[jl]], rows_v.at[b], gsem).wait()
                pltpu.async_copy(
                    rows_v.at[b], acc_sh.at[dst_v.at[jl]], ssem, add=True)

                @pl.when(jl >= 2)
                def _():
                    pltpu.make_async_copy(
                        rows_v.at[nx],
                        acc_sh.at[dst_v.at[jl - 2]], ssem).wait()

                @pl.when(jl + 2 < CK)
                def _():
                    pltpu.async_copy(
                        y_sh.at[src_v.at[jl + 2]], rows_v.at[nx], gsem)

        pltpu.make_async_copy(
            rows_v.at[2], acc_sh.at[dst_v.at[CK - 2]], ssem).wait()
        pltpu.make_async_copy(
            rows_v.at[3], acc_sh.at[dst_v.at[CK - 1]], ssem).wait()

    plsc.subcore_barrier()
    pltpu.sync_copy(acc_sh.at[pl.ds(row0, RPS)],
                    out_hbm.at[pl.ds(row0, RPS), pl.ds(col0, DH)])


@functools.partial(
    pl.kernel,
    out_type=jax.ShapeDtypeStruct((NC, NP, DC), jnp.float32),
    mesh=_mesh,
    compiler_params=pltpu.CompilerParams(use_tc_tiling_on_sc=False),
    scratch_types=[
        pltpu.VMEM((BPW, 128), jnp.int32),
        pltpu.VMEM((BPW, 128), jnp.int32),
        pltpu.VMEM((2, 128, DC), jnp.float32),
        pltpu.VMEM_SHARED((NP, DC), jnp.float32),
        pltpu.VMEM_SHARED((NP, DC), jnp.float32),
        pltpu.SemaphoreType.DMA,
    ],
)
def _sc_agg16(y_hbm, src_hbm, dst_hbm, z_hbm, out_hbm,
              src_v, dst_v, rows_v, y_sh, acc_sh, gsem):
    c = lax.axis_index("c")
    s = lax.axis_index("s")
    w = c * NS + s
    row0 = s * RPS
    pltpu.sync_copy(y_hbm.at[pl.ds(row0, RPS)], y_sh.at[pl.ds(row0, RPS)])
    pltpu.sync_copy(z_hbm, acc_sh.at[pl.ds(row0, RPS)])
    pltpu.sync_copy(src_hbm.at[w], src_v)
    pltpu.sync_copy(dst_hbm.at[w], dst_v)
    plsc.subcore_barrier()

    pltpu.async_copy(y_sh.at[src_v.at[0]], rows_v.at[0], gsem)
    pltpu.async_copy(y_sh.at[src_v.at[1]], rows_v.at[1], gsem)

    @pl.loop(0, BPW, step=2)
    def _(j):
        for b in range(2):
            jj = j + b
            pltpu.make_async_copy(
                y_sh.at[src_v.at[jj]], rows_v.at[b], gsem).wait()
            pltpu.sync_copy(rows_v.at[b], acc_sh.at[dst_v.at[jj]], add=True)

            @pl.when(jj + 2 < BPW)
            def _():
                pltpu.async_copy(
                    y_sh.at[src_v.at[jj + 2]], rows_v.at[b], gsem)

    plsc.subcore_barrier()
    pltpu.sync_copy(acc_sh.at[pl.ds(row0, RPS)],
                    out_hbm.at[c, pl.ds(row0, RPS)])


@functools.partial(
    pl.kernel,
    out_type=jax.ShapeDtypeStruct((NC, NP, DC), jnp.float32),
    mesh=_mesh,
    compiler_params=pltpu.CompilerParams(use_tc_tiling_on_sc=False),
    scratch_types=[
        pltpu.VMEM((BPW, 128), jnp.int32),
        pltpu.VMEM((128, DC), jnp.float32),
        pltpu.VMEM_SHARED((NP, DC), jnp.float32),
    ],
)
def _sc_deg(dst_hbm, ones_hbm, z_hbm, out_hbm, dst_v, ones_v, deg_sh):
    c = lax.axis_index("c")
    s = lax.axis_index("s")
    w = c * NS + s
    pltpu.sync_copy(z_hbm, deg_sh.at[pl.ds(s * RPS, RPS)])
    pltpu.sync_copy(ones_hbm, ones_v)
    pltpu.sync_copy(dst_hbm.at[w], dst_v)
    plsc.subcore_barrier()

    @pl.loop(0, BPW)
    def _(j):
        pltpu.sync_copy(ones_v, deg_sh.at[dst_v.at[j]], add=True)

    plsc.subcore_barrier()
    pltpu.sync_copy(deg_sh.at[pl.ds(s * RPS, RPS)],
                    out_hbm.at[c, pl.ds(s * RPS, RPS)])


def _tc_first_body(x_ref, degp_ref, w_ref, y_ref, dinv_ref):
    dp = degp_ref[...]
    deg = dp[0, :, 0] + dp[1, :, 0] + 1.0
    dinv = lax.rsqrt(deg)
    xw = jnp.dot(x_ref[...], w_ref[...], preferred_element_type=jnp.float32)
    y_ref[...] = dinv[:, None] * xw
    dinv_ref[...] = jnp.broadcast_to(dinv[:, None], (BR, D))


def _tc_first(x, degp, w1):
    return pl.pallas_call(
        _tc_first_body,
        grid=(GRID,),
        in_specs=[
            pl.BlockSpec((BR, D), lambda i: (i, 0)),
            pl.BlockSpec((NC, BR, DC), lambda i: (0, i, 0)),
            pl.BlockSpec((D, D), lambda i: (0, 0)),
        ],
        out_specs=[
            pl.BlockSpec((BR, D), lambda i: (i, 0)),
            pl.BlockSpec((BR, D), lambda i: (i, 0)),
        ],
        out_shape=[
            jax.ShapeDtypeStruct((NP, D), jnp.float32),
            jax.ShapeDtypeStruct((NP, D), jnp.float32),
        ],
    )(x, degp, w1)


def _tc_mid_body(agg_ref, y_ref, dinv_ref, b_ref, w_ref, o_ref, nout):
    dinv = dinv_ref[...]
    h = dinv * (agg_ref[...] + y_ref[...]) + b_ref[...]
    h = jnp.maximum(h, 0.0)
    hw = jnp.dot(h, w_ref[...], preferred_element_type=jnp.float32)
    o_ref[...] = dinv[:, :nout] * hw


def _tc_mid(agg, y, dinv, b, w):
    nout = w.shape[1]
    return pl.pallas_call(
        functools.partial(_tc_mid_body, nout=nout),
        grid=(GRID,),
        in_specs=[
            pl.BlockSpec((BR, D), lambda i: (i, 0)),
            pl.BlockSpec((BR, D), lambda i: (i, 0)),
            pl.BlockSpec((BR, D), lambda i: (i, 0)),
            pl.BlockSpec((1, D), lambda i: (0, 0)),
            pl.BlockSpec((D, nout), lambda i: (0, 0)),
        ],
        out_specs=pl.BlockSpec((BR, nout), lambda i: (i, 0)),
        out_shape=jax.ShapeDtypeStruct((NP, nout), jnp.float32),
    )(agg, y, dinv, b, w)


def _tc_pool_body(aggp_ref, y_ref, dinv_ref, b_ref, batch_ref, o_ref,
                  sums_ref, cnts_ref):
    i = pl.program_id(0)
    ag = aggp_ref[...]
    dinv = dinv_ref[...][:, :DC]
    h4 = dinv * (ag[0] + ag[1] + y_ref[...]) + b_ref[...]
    gid = jax.lax.broadcasted_iota(jnp.int32, (BR, G), 1)
    onehot = (batch_ref[...] == gid).astype(jnp.float32)
    part_s = lax.dot_general(onehot, h4, (((0,), (0,)), ((), ())),
                             preferred_element_type=jnp.float32)
    part_c = jnp.sum(onehot, axis=0)

    @pl.when(i == 0)
    def _():
        sums_ref[...] = part_s
        cnts_ref[...] = jnp.broadcast_to(part_c[:, None], (G, DC))

    @pl.when(i > 0)
    def _():
        sums_ref[...] += part_s
        cnts_ref[...] += jnp.broadcast_to(part_c[:, None], (G, DC))

    @pl.when(i == GRID - 1)
    def _():
        o_ref[...] = sums_ref[...] / jnp.maximum(cnts_ref[...], 1.0)


def _tc_pool(aggp, y, dinv, b, batch2d):
    return pl.pallas_call(
        _tc_pool_body,
        grid=(GRID,),
        in_specs=[
            pl.BlockSpec((NC, BR, DC), lambda i: (0, i, 0)),
            pl.BlockSpec((BR, DC), lambda i: (i, 0)),
            pl.BlockSpec((BR, D), lambda i: (i, 0)),
            pl.BlockSpec((1, DC), lambda i: (0, 0)),
            pl.BlockSpec((BR, 1), lambda i: (i, 0)),
        ],
        out_specs=pl.BlockSpec((G, DC), lambda i: (0, 0)),
        out_shape=jax.ShapeDtypeStruct((G, DC), jnp.float32),
        scratch_shapes=[
            pltpu.VMEM((G, DC), jnp.float32),
            pltpu.VMEM((G, DC), jnp.float32),
        ],
    )(aggp, y, dinv, b, batch2d)


def kernel(x, edge_index, batch, W1, b1, W2, b2, W3, b3, W4, b4):
    f32 = jnp.float32
    src = edge_index[0]
    dst = edge_index[1]
    npad = EPAD - E
    srcf = jnp.concatenate([src, jnp.zeros((npad,), jnp.int32)])
    dstf = jnp.concatenate([dst, jnp.full((npad,), DUMMY, jnp.int32)])
    srcA = srcf.reshape(NS, NQ, CK, 128)
    dstA = dstf.reshape(NS, NQ, CK, 128)
    srcB = srcf.reshape(NW, BPW, 128)
    dstB = dstf.reshape(NW, BPW, 128)

    xp = jnp.zeros((NP, D), f32).at[:N].set(x)
    batch2d = jnp.full((NP, 1), -1, jnp.int32).at[:N, 0].set(batch)

    w4p = jnp.zeros((D, DC), f32).at[:, :10].set(W4)
    b4p = jnp.zeros((1, DC), f32).at[0, :10].set(b4)

    z16 = jnp.zeros((RPS, DC), f32)
    z64 = jnp.zeros((RPS, DH), f32)
    ones16 = jnp.ones((128, DC), f32)

    degp = _sc_deg(dstB, ones16, z16)
    y1, dinv = _tc_first(xp, degp, W1)

    agg1 = _sc_agg128(y1, srcA, dstA, z64)
    y2 = _tc_mid(agg1, y1, dinv, b1.reshape(1, D), W2)

    agg2 = _sc_agg128(y2, srcA, dstA, z64)
    y3 = _tc_mid(agg2, y2, dinv, b2.reshape(1, D), W3)

    agg3 = _sc_agg128(y3, srcA, dstA, z64)
    y4 = _tc_mid(agg3, y3, dinv, b3.reshape(1, D), w4p)

    agg4 = _sc_agg16(y4, srcB, dstB, z16)
    out = _tc_pool(agg4, y4, dinv, b4p, batch2d)
    return out[:, :10]


# speedup vs baseline: 3.3628x; 1.0151x over previous
import functools

import jax
import jax.numpy as jnp
from jax import lax
from jax.experimental import pallas as pl
from jax.experimental.pallas import tpu as pltpu
from jax.experimental.pallas import tpu_sc as plsc

N = 10000
NP = 10240
E = 320000
D = 128
DH = 64
DC = 16
G = 64
NC = 2
NS = 16
NW = NC * NS
BPT = 160
BPW = 80
NQ = 4
CK = BPT // NQ
EPAD = NS * BPT * 128
DUMMY = N
BR = 2048
GRID = NP // BR
RPS = NP // NS

_mesh = plsc.VectorSubcoreMesh(core_axis_name="c", subcore_axis_name="s",
                               num_cores=NC, num_subcores=NS)


@functools.partial(
    pl.kernel,
    out_type=jax.ShapeDtypeStruct((NP, D), jnp.float32),
    mesh=_mesh,
    compiler_params=pltpu.CompilerParams(use_tc_tiling_on_sc=False),
    scratch_types=[
        pltpu.VMEM((CK, 128), jnp.int32),
        pltpu.VMEM((CK, 128), jnp.int32),
        pltpu.VMEM((4, 128, DH), jnp.float32),
        pltpu.VMEM_SHARED((NP, DH), jnp.float32),
        pltpu.VMEM_SHARED((NP, DH), jnp.float32),
        pltpu.SemaphoreType.DMA,
        pltpu.SemaphoreType.DMA,
    ],
)
def _sc_agg128(y_hbm, src_hbm, dst_hbm, z_hbm, out_hbm,
               src_v, dst_v, rows_v, y_sh, acc_sh, gsem, ssem):
    c = lax.axis_index("c")
    s = lax.axis_index("s")
    row0 = s * RPS
    col0 = c * DH
    pltpu.sync_copy(y_hbm.at[pl.ds(row0, RPS), pl.ds(col0, DH)],
                    y_sh.at[pl.ds(row0, RPS)])
    pltpu.sync_copy(z_hbm, acc_sh.at[pl.ds(row0, RPS)])
    plsc.subcore_barrier()

    @pl.loop(0, NQ)
    def _(q):
        pltpu.sync_copy(src_hbm.at[s, q], src_v)
        pltpu.sync_copy(dst_hbm.at[s, q], dst_v)
        pltpu.async_copy(y_sh.at[src_v.at[0]], rows_v.at[0], gsem)
        pltpu.async_copy(y_sh.at[src_v.at[1]], rows_v.at[1], gsem)

        @pl.loop(0, CK, step=4)
        def _(r):
            for b in range(4):
                jl = r + b
                nx = (b + 2) % 4
                pltpu.make_async_copy(
                    y_sh.at[src_v.at[jl]], rows_v.at[b], gsem).wait()
                pltpu.async_copy(
                    rows_v.at[b], acc_sh.at[dst_v.at[jl]], ssem, add=True)

                @pl.when(jl >= 2)
                def _():
                    pltpu.make_async_copy(
                        rows_v.at[nx],
                        acc_sh.at[dst_v.at[jl - 2]], ssem).wait()

                @pl.when(jl + 2 < CK)
                def _():
                    pltpu.async_copy(
                        y_sh.at[src_v.at[jl + 2]], rows_v.at[nx], gsem)

        pltpu.make_async_copy(
            rows_v.at[2], acc_sh.at[dst_v.at[CK - 2]], ssem).wait()
        pltpu.make_async_copy(
            rows_v.at[3], acc_sh.at[dst_v.at[CK - 1]], ssem).wait()

    plsc.subcore_barrier()
    pltpu.sync_copy(acc_sh.at[pl.ds(row0, RPS)],
                    out_hbm.at[pl.ds(row0, RPS), pl.ds(col0, DH)])


@functools.partial(
    pl.kernel,
    out_type=jax.ShapeDtypeStruct((NC, NP, D), jnp.float32),
    mesh=_mesh,
    compiler_params=pltpu.CompilerParams(use_tc_tiling_on_sc=False),
    scratch_types=[
        pltpu.VMEM((BPW, 128), jnp.int32),
        pltpu.VMEM((BPW, 128), jnp.int32),
        pltpu.VMEM((2, 128, DC), jnp.float32),
        pltpu.VMEM_SHARED((NP, DC), jnp.float32),
        pltpu.VMEM_SHARED((NP, DC), jnp.float32),
        pltpu.SemaphoreType.DMA,
    ],
)
def _sc_agg16(y_hbm, src_hbm, dst_hbm, z_hbm, out_hbm,
              src_v, dst_v, rows_v, y_sh, acc_sh, gsem):
    c = lax.axis_index("c")
    s = lax.axis_index("s")
    w = c * NS + s
    row0 = s * RPS
    pltpu.sync_copy(y_hbm.at[pl.ds(row0, RPS), pl.ds(0, DC)],
                    y_sh.at[pl.ds(row0, RPS)])
    pltpu.sync_copy(z_hbm, acc_sh.at[pl.ds(row0, RPS)])
    pltpu.sync_copy(src_hbm.at[w], src_v)
    pltpu.sync_copy(dst_hbm.at[w], dst_v)
    plsc.subcore_barrier()

    pltpu.async_copy(y_sh.at[src_v.at[0]], rows_v.at[0], gsem)
    pltpu.async_copy(y_sh.at[src_v.at[1]], rows_v.at[1], gsem)

    @pl.loop(0, BPW, step=2)
    def _(j):
        for b in range(2):
            jj = j + b
            pltpu.make_async_copy(
                y_sh.at[src_v.at[jj]], rows_v.at[b], gsem).wait()
            pltpu.sync_copy(rows_v.at[b], acc_sh.at[dst_v.at[jj]], add=True)

            @pl.when(jj + 2 < BPW)
            def _():
                pltpu.async_copy(
                    y_sh.at[src_v.at[jj + 2]], rows_v.at[b], gsem)

    plsc.subcore_barrier()
    pltpu.sync_copy(acc_sh.at[pl.ds(row0, RPS)],
                    out_hbm.at[c, pl.ds(row0, RPS), pl.ds(0, DC)])


@functools.partial(
    pl.kernel,
    out_type=jax.ShapeDtypeStruct((NC, NP, D), jnp.float32),
    mesh=_mesh,
    compiler_params=pltpu.CompilerParams(use_tc_tiling_on_sc=False),
    scratch_types=[
        pltpu.VMEM((BPW, 128), jnp.int32),
        pltpu.VMEM((128, DC), jnp.float32),
        pltpu.VMEM_SHARED((NP, DC), jnp.float32),
    ],
)
def _sc_deg(dst_hbm, ones_hbm, z_hbm, out_hbm, dst_v, ones_v, deg_sh):
    c = lax.axis_index("c")
    s = lax.axis_index("s")
    w = c * NS + s
    pltpu.sync_copy(z_hbm, deg_sh.at[pl.ds(s * RPS, RPS)])
    pltpu.sync_copy(ones_hbm, ones_v)
    pltpu.sync_copy(dst_hbm.at[w], dst_v)
    plsc.subcore_barrier()

    @pl.loop(0, BPW)
    def _(j):
        pltpu.sync_copy(ones_v, deg_sh.at[dst_v.at[j]], add=True)

    plsc.subcore_barrier()
    pltpu.sync_copy(deg_sh.at[pl.ds(s * RPS, RPS)],
                    out_hbm.at[c, pl.ds(s * RPS, RPS), pl.ds(0, DC)])


def _tc_first_body(x_ref, degp_ref, w_ref, y_ref, dinv_ref):
    dp = degp_ref[...]
    deg = dp[0, :, 0] + dp[1, :, 0] + 1.0
    dinv = lax.rsqrt(deg)
    xw = jnp.dot(x_ref[...], w_ref[...], preferred_element_type=jnp.float32)
    y_ref[...] = dinv[:, None] * xw
    dinv_ref[...] = jnp.broadcast_to(dinv[:, None], (BR, D))


def _tc_first(x, degp, w1):
    return pl.pallas_call(
        _tc_first_body,
        grid=(GRID,),
        in_specs=[
            pl.BlockSpec((BR, D), lambda i: (i, 0)),
            pl.BlockSpec((NC, BR, D), lambda i: (0, i, 0)),
            pl.BlockSpec((D, D), lambda i: (0, 0)),
        ],
        out_specs=[
            pl.BlockSpec((BR, D), lambda i: (i, 0)),
            pl.BlockSpec((BR, D), lambda i: (i, 0)),
        ],
        out_shape=[
            jax.ShapeDtypeStruct((NP, D), jnp.float32),
            jax.ShapeDtypeStruct((NP, D), jnp.float32),
        ],
    )(x, degp, w1)


def _tc_mid_body(agg_ref, y_ref, dinv_ref, b_ref, w_ref, o_ref, nout):
    dinv = dinv_ref[...]
    h = dinv * (agg_ref[...] + y_ref[...]) + b_ref[...]
    h = jnp.maximum(h, 0.0)
    hw = jnp.dot(h, w_ref[...], preferred_element_type=jnp.float32)
    o_ref[...] = dinv[:, :nout] * hw


def _tc_mid(agg, y, dinv, b, w):
    nout = w.shape[1]
    return pl.pallas_call(
        functools.partial(_tc_mid_body, nout=nout),
        grid=(GRID,),
        in_specs=[
            pl.BlockSpec((BR, D), lambda i: (i, 0)),
            pl.BlockSpec((BR, D), lambda i: (i, 0)),
            pl.BlockSpec((BR, D), lambda i: (i, 0)),
            pl.BlockSpec((1, D), lambda i: (0, 0)),
            pl.BlockSpec((D, nout), lambda i: (0, 0)),
        ],
        out_specs=pl.BlockSpec((BR, nout), lambda i: (i, 0)),
        out_shape=jax.ShapeDtypeStruct((NP, nout), jnp.float32),
    )(agg, y, dinv, b, w)


def _tc_pool_body(aggp_ref, y_ref, dinv_ref, b_ref, batch_ref, o_ref,
                  sums_ref, cnts_ref):
    i = pl.program_id(0)
    ag = aggp_ref[...]
    dinv = dinv_ref[...][:, :DC]
    h4 = (dinv * (ag[0, :, :DC] + ag[1, :, :DC] + y_ref[...][:, :DC])
          + b_ref[...])
    gid = jax.lax.broadcasted_iota(jnp.int32, (BR, G), 1)
    onehot = (batch_ref[...] == gid).astype(jnp.float32)
    part_s = lax.dot_general(onehot, h4, (((0,), (0,)), ((), ())),
                             preferred_element_type=jnp.float32)
    part_c = jnp.sum(onehot, axis=0)

    @pl.when(i == 0)
    def _():
        sums_ref[...] = part_s
        cnts_ref[...] = jnp.broadcast_to(part_c[:, None], (G, DC))

    @pl.when(i > 0)
    def _():
        sums_ref[...] += part_s
        cnts_ref[...] += jnp.broadcast_to(part_c[:, None], (G, DC))

    @pl.when(i == GRID - 1)
    def _():
        o_ref[...] = sums_ref[...] / jnp.maximum(cnts_ref[...], 1.0)


def _tc_pool(aggp, y, dinv, b, batch2d):
    return pl.pallas_call(
        _tc_pool_body,
        grid=(GRID,),
        in_specs=[
            pl.BlockSpec((NC, BR, D), lambda i: (0, i, 0)),
            pl.BlockSpec((BR, D), lambda i: (i, 0)),
            pl.BlockSpec((BR, D), lambda i: (i, 0)),
            pl.BlockSpec((1, DC), lambda i: (0, 0)),
            pl.BlockSpec((BR, 1), lambda i: (i, 0)),
        ],
        out_specs=pl.BlockSpec((G, DC), lambda i: (0, 0)),
        out_shape=jax.ShapeDtypeStruct((G, DC), jnp.float32),
        scratch_shapes=[
            pltpu.VMEM((G, DC), jnp.float32),
            pltpu.VMEM((G, DC), jnp.float32),
        ],
    )(aggp, y, dinv, b, batch2d)


def kernel(x, edge_index, batch, W1, b1, W2, b2, W3, b3, W4, b4):
    f32 = jnp.float32
    src = edge_index[0]
    dst = edge_index[1]
    npad = EPAD - E
    srcf = jnp.concatenate([src, jnp.zeros((npad,), jnp.int32)])
    dstf = jnp.concatenate([dst, jnp.full((npad,), DUMMY, jnp.int32)])
    srcA = srcf.reshape(NS, NQ, CK, 128)
    dstA = dstf.reshape(NS, NQ, CK, 128)
    srcB = srcf.reshape(NW, BPW, 128)
    dstB = dstf.reshape(NW, BPW, 128)

    xp = jnp.zeros((NP, D), f32).at[:N].set(x)
    batch2d = jnp.full((NP, 1), -1, jnp.int32).at[:N, 0].set(batch)

    w4p = jnp.zeros((D, D), f32).at[:, :10].set(W4)
    b4p = jnp.zeros((1, DC), f32).at[0, :10].set(b4)

    z16 = jnp.zeros((RPS, DC), f32)
    z64 = jnp.zeros((RPS, DH), f32)
    ones16 = jnp.ones((128, DC), f32)

    degp = _sc_deg(dstB, ones16, z16)
    y1, dinv = _tc_first(xp, degp, W1)

    agg1 = _sc_agg128(y1, srcA, dstA, z64)
    y2 = _tc_mid(agg1, y1, dinv, b1.reshape(1, D), W2)

    agg2 = _sc_agg128(y2, srcA, dstA, z64)
    y3 = _tc_mid(agg2, y2, dinv, b2.reshape(1, D), W3)

    agg3 = _sc_agg128(y3, srcA, dstA, z64)
    y4 = _tc_mid(agg3, y3, dinv, b3.reshape(1, D), w4p)

    agg4 = _sc_agg16(y4, srcB, dstB, z16)
    out = _tc_pool(agg4, y4, dinv, b4p, batch2d)
    return out[:, :10]
